# baseline (speedup 1.0000x reference)
import functools

import jax
import jax.numpy as jnp
from jax import lax
from jax.experimental import pallas as pl
from jax.experimental.pallas import tpu as pltpu
from jax.experimental.pallas import tpu_sc as plsc

NC = 2
NS = 16
NW = NC * NS
L = 16

B = 16384
F = 26
EMB = 16
NUMERIC = 13
V = 1_000_000

ROWS_PER_W = B // NW
CHUNK = 64
NCHUNK = ROWS_PER_W // CHUNK
IDX_PER_CHUNK = CHUNK * F
NG = CHUNK // L


def _sc_fm_stats(inputs_t, vt, wt):
    mesh = plsc.VectorSubcoreMesh(
        core_axis_name="c", subcore_axis_name="s",
        num_cores=NC, num_subcores=NS)

    @functools.partial(
        pl.kernel,
        out_type=(
            jax.ShapeDtypeStruct((EMB, B), jnp.float32),
            jax.ShapeDtypeStruct((EMB, B), jnp.float32),
            jax.ShapeDtypeStruct((B,), jnp.float32),
        ),
        mesh=mesh,
        compiler_params=pltpu.CompilerParams(use_tc_tiling_on_sc=False),
        scratch_types=(
            pltpu.VMEM((F, ROWS_PER_W), jnp.float32),
            pltpu.VMEM((F, ROWS_PER_W), jnp.int32),
            pltpu.VMEM((2, EMB, IDX_PER_CHUNK), jnp.float32),
            pltpu.VMEM((2, IDX_PER_CHUNK), jnp.float32),
            pltpu.VMEM((2, EMB, CHUNK), jnp.float32),
            pltpu.VMEM((2, EMB, CHUNK), jnp.float32),
            pltpu.VMEM((2, CHUNK), jnp.float32),
            pltpu.SemaphoreType.DMA,
            pltpu.SemaphoreType.DMA,
        ),
    )
    def sc_kernel(idx_hbm, vt_hbm, wt_hbm, s_out, q_out, w_out,
                  idxf, idx_all, vvals, wvals, s_stage, q_stage, w_stage,
                  gsem, osem):
        wid = lax.axis_index("s") * NC + lax.axis_index("c")
        row0 = wid * ROWS_PER_W
        pltpu.sync_copy(
            idx_hbm.at[pl.ds(0, F), pl.ds(row0, ROWS_PER_W)], idxf)

        def conv_body(i, carry):
            for k in range(ROWS_PER_W // L):
                vf = idxf[i, pl.ds(k * L, L)]
                idx_all[i, pl.ds(k * L, L)] = vf.astype(jnp.int32)
            return carry

        lax.fori_loop(0, F, conv_body, 0)

        def issue_gathers(c):
            slot = c % 2

            def j_body(j, carry):
                irow = idx_all.at[j].at[pl.ds(c * CHUNK, CHUNK)]
                for k in range(EMB):
                    dst = vvals.at[slot].at[k].at[pl.ds(j * CHUNK, CHUNK)]
                    pltpu.async_copy(vt_hbm.at[k].at[irow], dst, gsem)
                dst_w = wvals.at[slot].at[pl.ds(j * CHUNK, CHUNK)]
                pltpu.async_copy(wt_hbm.at[0].at[irow], dst_w, gsem)
                return carry

            lax.fori_loop(0, F, j_body, 0)

        def wait_gathers(c):
            slot = c % 2
            pltpu.make_async_copy(
                vt_hbm.at[pl.ds(0, EMB), pl.ds(0, IDX_PER_CHUNK)],
                vvals.at[slot], gsem).wait()
            pltpu.make_async_copy(
                wt_hbm.at[0].at[pl.ds(0, IDX_PER_CHUNK)],
                wvals.at[slot], gsem).wait()

        def compute(c):
            slot = c % 2
            wv = wvals.at[slot]

            for g in range(NG):
                zero = jnp.zeros((L,), jnp.float32)

                def j_body(j, accs):
                    ss, qq = accs
                    ss, qq = list(ss), list(qq)
                    for k in range(EMB):
                        v = vvals[slot, k, pl.ds(j * CHUNK + g * L, L)]
                        ss[k] = ss[k] + v
                        qq[k] = qq[k] + v * v
                    return tuple(ss), tuple(qq)

                ss, qq = lax.fori_loop(
                    0, F, j_body, ((zero,) * EMB, (zero,) * EMB))
                for k in range(EMB):
                    s_stage[slot, k, pl.ds(g * L, L)] = ss[k]
                    q_stage[slot, k, pl.ds(g * L, L)] = qq[k]

            def fo_body(j, accs):
                return tuple(
                    accs[g] + wv[pl.ds(j * CHUNK + g * L, L)]
                    for g in range(NG))

            zero = jnp.zeros((L,), jnp.float32)
            accs = lax.fori_loop(0, F, fo_body, (zero,) * NG)
            for g in range(NG):
                w_stage[slot, pl.ds(g * L, L)] = accs[g]

        def issue_out(c):
            slot = c % 2
            ob = wid * ROWS_PER_W + c * CHUNK
            return [
                pltpu.async_copy(
                    s_stage.at[slot],
                    s_out.at[pl.ds(0, EMB), pl.ds(ob, CHUNK)], osem),
                pltpu.async_copy(
                    q_stage.at[slot],
                    q_out.at[pl.ds(0, EMB), pl.ds(ob, CHUNK)], osem),
                pltpu.async_copy(w_stage.at[slot], w_out.at[pl.ds(ob, CHUNK)], osem),
            ]

        issue_gathers(0)
        od = {}
        for c in range(NCHUNK):
            if c + 1 < NCHUNK:
                issue_gathers(c + 1)
            wait_gathers(c)
            if c - 2 in od:
                for d in od.pop(c - 2):
                    d.wait()
            compute(c)
            od[c] = issue_out(c)
        for descs in od.values():
            for d in descs:
                d.wait()

    return sc_kernel(inputs_t, vt, wt)


def _tc_combine(xt_ref, vn_ref, wn_ref, b_ref, s_ref, q_ref, w_ref, o_ref):
    xt = xt_ref[...]
    dn = (((0,), (0,)), ((), ()))
    sn = lax.dot_general(vn_ref[...], xt, dn,
                         preferred_element_type=jnp.float32)
    qn = lax.dot_general(vn_ref[...] * vn_ref[...], xt * xt, dn,
                         preferred_element_type=jnp.float32)
    st = s_ref[...] + sn
    ss = st * st - q_ref[...] - qn
    second = 0.5 * jnp.sum(ss, axis=0, keepdims=True)
    first = w_ref[...] + lax.dot_general(
        wn_ref[...], xt, dn, preferred_element_type=jnp.float32)
    o_ref[...] = first + second + b_ref[0, 0]


def kernel(inputs, w_one_hot, w_numeric, v_one_hot, v_numeric, b):
    inputs_t = inputs.T
    st, qt, wsum = _sc_fm_stats(inputs_t, v_one_hot.T, w_one_hot.T)

    numeric_t = inputs_t[F:]
    blk = 2048
    out1 = pl.pallas_call(
        _tc_combine,
        grid=(B // blk,),
        in_specs=[
            pl.BlockSpec((NUMERIC, blk), lambda i: (0, i)),
            pl.BlockSpec((NUMERIC, EMB), lambda i: (0, 0)),
            pl.BlockSpec((NUMERIC, 1), lambda i: (0, 0)),
            pl.BlockSpec((1, 1), lambda i: (0, 0)),
            pl.BlockSpec((EMB, blk), lambda i: (0, i)),
            pl.BlockSpec((EMB, blk), lambda i: (0, i)),
            pl.BlockSpec((1, blk), lambda i: (0, i)),
        ],
        out_specs=pl.BlockSpec((1, blk), lambda i: (0, i)),
        out_shape=jax.ShapeDtypeStruct((1, B), jnp.float32),
    )(numeric_t, v_numeric, w_numeric, b.reshape(1, 1),
      st, qt, wsum.reshape(1, B))
    return out1.reshape(B, 1)


# speedup vs baseline: 1.9106x; 1.9106x over previous
import functools

import jax
import jax.numpy as jnp
from jax import lax
from jax.experimental import pallas as pl
from jax.experimental.pallas import tpu as pltpu
from jax.experimental.pallas import tpu_sc as plsc

NC = 2
NS = 16
NW = NC * NS
L = 16

B = 16384
F = 26
EMB = 16
NUMERIC = 13
V = 1_000_000

ROWS_PER_W = B // NW
CHUNK = 64
NCHUNK = ROWS_PER_W // CHUNK
IDX_PER_CHUNK = CHUNK * F


def _tree_add(vs):
    while len(vs) > 1:
        nxt = [vs[i] + vs[i + 1] for i in range(0, len(vs) - 1, 2)]
        if len(vs) % 2:
            nxt.append(vs[-1])
        vs = nxt
    return vs[0]


def _sc_fm_stats(inputs_t, v_tab, w_tab):
    mesh = plsc.VectorSubcoreMesh(
        core_axis_name="c", subcore_axis_name="s",
        num_cores=NC, num_subcores=NS)

    @functools.partial(
        pl.kernel,
        out_type=(
            jax.ShapeDtypeStruct((B, EMB), jnp.float32),
            jax.ShapeDtypeStruct((B, EMB), jnp.float32),
            jax.ShapeDtypeStruct((B,), jnp.float32),
        ),
        mesh=mesh,
        compiler_params=pltpu.CompilerParams(use_tc_tiling_on_sc=False),
        scratch_types=(
            pltpu.VMEM((F, ROWS_PER_W), jnp.float32),
            pltpu.VMEM((F, ROWS_PER_W), jnp.int32),
            pltpu.VMEM((2, IDX_PER_CHUNK, EMB), jnp.float32),
            pltpu.VMEM((2, IDX_PER_CHUNK), jnp.float32),
            pltpu.VMEM((2, CHUNK, EMB), jnp.float32),
            pltpu.VMEM((2, CHUNK, EMB), jnp.float32),
            pltpu.VMEM((2, CHUNK), jnp.float32),
            pltpu.SemaphoreType.DMA,
            pltpu.SemaphoreType.DMA,
        ),
    )
    def sc_kernel(idx_hbm, v_hbm, w_hbm, s_out, q_out, w_out,
                  idxf, idx_all, vrows, wvals, s_stage, q_stage, w_stage,
                  gsem, osem):
        wid = lax.axis_index("s") * NC + lax.axis_index("c")
        row0 = wid * ROWS_PER_W
        pltpu.sync_copy(
            idx_hbm.at[pl.ds(0, F), pl.ds(row0, ROWS_PER_W)], idxf)

        def conv_body(i, carry):
            for k in range(ROWS_PER_W // L):
                vf = idxf[i, pl.ds(k * L, L)]
                idx_all[i, pl.ds(k * L, L)] = vf.astype(jnp.int32)
            return carry

        lax.fori_loop(0, F, conv_body, 0)

        def issue_gathers(c):
            slot = c % 2
            descs = []
            for j in range(F):
                irow = idx_all.at[j].at[pl.ds(c * CHUNK, CHUNK)]
                dst_v = vrows.at[slot].at[pl.ds(j * CHUNK, CHUNK)]
                descs.append(pltpu.async_copy(v_hbm.at[irow], dst_v, gsem))
                dst_w = wvals.at[slot].at[pl.ds(j * CHUNK, CHUNK)]
                descs.append(pltpu.async_copy(w_hbm.at[irow], dst_w, gsem))
            return descs

        def compute(c):
            slot = c % 2
            vr = vrows.at[slot]
            wv = wvals.at[slot]

            def row_body(r, carry):
                vs = [vr[j * CHUNK + r, :] for j in range(F)]
                s_stage[slot, r, :] = _tree_add(vs)
                q_stage[slot, r, :] = _tree_add([v * v for v in vs])
                return carry

            lax.fori_loop(0, CHUNK, row_body, 0)

            def fo_body(j, accs):
                return tuple(
                    accs[k] + wv[pl.ds(j * CHUNK + k * L, L)]
                    for k in range(CHUNK // L))

            zero = jnp.zeros((L,), jnp.float32)
            accs = lax.fori_loop(0, F, fo_body, (zero,) * (CHUNK // L))
            for k in range(CHUNK // L):
                w_stage[slot, pl.ds(k * L, L)] = accs[k]

        def issue_out(c):
            slot = c % 2
            ob = wid * ROWS_PER_W + c * CHUNK
            return [
                pltpu.async_copy(s_stage.at[slot], s_out.at[pl.ds(ob, CHUNK)], osem),
                pltpu.async_copy(q_stage.at[slot], q_out.at[pl.ds(ob, CHUNK)], osem),
                pltpu.async_copy(w_stage.at[slot], w_out.at[pl.ds(ob, CHUNK)], osem),
            ]

        gd = {0: issue_gathers(0)}
        od = {}
        for c in range(NCHUNK):
            if c + 1 < NCHUNK:
                gd[c + 1] = issue_gathers(c + 1)
            for d in gd.pop(c):
                d.wait()
            if c - 2 in od:
                for d in od.pop(c - 2):
                    d.wait()
            compute(c)
            od[c] = issue_out(c)
        for descs in od.values():
            for d in descs:
                d.wait()

    return sc_kernel(inputs_t, v_tab, w_tab)


def _tc_combine(n2_ref, vne_ref, vn2e_ref, wne_ref, m_ref, b_ref,
                s_ref, q_ref, wv_ref, o_ref):
    x = n2_ref[...]
    sn = jnp.dot(x, vne_ref[...], preferred_element_type=jnp.float32)
    qn = jnp.dot(x * x, vn2e_ref[...], preferred_element_type=jnp.float32)
    st = s_ref[...] + sn
    ss = st * st - q_ref[...] - qn
    second = 0.5 * jnp.dot(ss, m_ref[...], preferred_element_type=jnp.float32)
    first = wv_ref[...] + jnp.dot(x, wne_ref[...], preferred_element_type=jnp.float32)
    o_ref[...] = first + second + b_ref[0, 0]


def _tc_transpose(x_ref, i_ref, o_ref):
    dn = (((0,), (0,)), ((), ()))
    for t in range(8):
        o_ref[0, pl.ds(t * 500, 500), :] = lax.dot_general(
            x_ref[:, t, :], i_ref[...], dn,
            preferred_element_type=jnp.float32)


def _transpose_table(vt):
    vt3 = vt.reshape(EMB, 2000, 500)
    out = pl.pallas_call(
        _tc_transpose,
        grid=(250,),
        in_specs=[
            pl.BlockSpec((EMB, 8, 500), lambda i: (0, i, 0)),
            pl.BlockSpec((EMB, EMB), lambda i: (0, 0)),
        ],
        out_specs=pl.BlockSpec((1, 4000, EMB), lambda i: (i, 0, 0)),
        out_shape=jax.ShapeDtypeStruct((250, 4000, EMB), jnp.float32),
    )(vt3, jnp.eye(EMB, dtype=jnp.float32))
    return out.reshape(V, EMB)


def kernel(inputs, w_one_hot, w_numeric, v_one_hot, v_numeric, b):
    inputs_t = inputs.T
    v_rm = _transpose_table(v_one_hot.T)
    s, q, wsum = _sc_fm_stats(inputs_t, v_rm, w_one_hot.reshape(V))

    n2 = inputs[:, F:].reshape(B // 8, 8 * NUMERIC)
    eye8 = jnp.eye(8, dtype=jnp.float32)
    vne = jnp.kron(eye8, v_numeric)
    vn2e = jnp.kron(eye8, v_numeric * v_numeric)
    wne = jnp.kron(eye8, w_numeric)
    m = jnp.kron(eye8, jnp.ones((EMB, 1), jnp.float32))
    s_v = s.reshape(B * EMB // 128, 128)
    q_v = q.reshape(B * EMB // 128, 128)
    w_v = wsum.reshape(B // 8, 8)

    blk = 256
    g = (B // 8) // blk
    out8 = pl.pallas_call(
        _tc_combine,
        grid=(g,),
        in_specs=[
            pl.BlockSpec((blk, 8 * NUMERIC), lambda i: (i, 0)),
            pl.BlockSpec((8 * NUMERIC, 128), lambda i: (0, 0)),
            pl.BlockSpec((8 * NUMERIC, 128), lambda i: (0, 0)),
            pl.BlockSpec((8 * NUMERIC, 8), lambda i: (0, 0)),
            pl.BlockSpec((128, 8), lambda i: (0, 0)),
            pl.BlockSpec((1, 1), lambda i: (0, 0)),
            pl.BlockSpec((blk, 128), lambda i: (i, 0)),
            pl.BlockSpec((blk, 128), lambda i: (i, 0)),
            pl.BlockSpec((blk, 8), lambda i: (i, 0)),
        ],
        out_specs=pl.BlockSpec((blk, 8), lambda i: (i, 0)),
        out_shape=jax.ShapeDtypeStruct((B // 8, 8), jnp.float32),
    )(n2, vne, vn2e, wne, m, b.reshape(1, 1), s_v, q_v, w_v)
    return out8.reshape(B, 1)


# speedup vs baseline: 3.2056x; 1.6779x over previous
import functools

import jax
import jax.numpy as jnp
from jax import lax
from jax.experimental import pallas as pl
from jax.experimental.pallas import tpu as pltpu
from jax.experimental.pallas import tpu_sc as plsc

NC = 2
NS = 16
NW = NC * NS
L = 16

B = 16384
F = 26
EMB = 16
NUMERIC = 13
V = 1_000_000

ROWS_PER_W = B // NW
CHUNK = 64
NCHUNK = ROWS_PER_W // CHUNK
IDX_PER_CHUNK = CHUNK * F


def _tree_add(vs):
    while len(vs) > 1:
        nxt = [vs[i] + vs[i + 1] for i in range(0, len(vs) - 1, 2)]
        if len(vs) % 2:
            nxt.append(vs[-1])
        vs = nxt
    return vs[0]


def _sc_fm_stats(inputs_t, v_tab, w_tab):
    mesh = plsc.VectorSubcoreMesh(
        core_axis_name="c", subcore_axis_name="s",
        num_cores=NC, num_subcores=NS)

    @functools.partial(
        pl.kernel,
        out_type=(
            jax.ShapeDtypeStruct((B, EMB), jnp.float32),
            jax.ShapeDtypeStruct((B, EMB), jnp.float32),
            jax.ShapeDtypeStruct((B,), jnp.float32),
        ),
        mesh=mesh,
        compiler_params=pltpu.CompilerParams(use_tc_tiling_on_sc=False),
        scratch_types=(
            pltpu.VMEM((F, ROWS_PER_W), jnp.float32),
            pltpu.VMEM((F, ROWS_PER_W), jnp.int32),
            pltpu.VMEM((2, IDX_PER_CHUNK, EMB), jnp.float32),
            pltpu.VMEM((2, IDX_PER_CHUNK), jnp.float32),
            pltpu.VMEM((2, CHUNK, EMB), jnp.float32),
            pltpu.VMEM((2, CHUNK, EMB), jnp.float32),
            pltpu.VMEM((2, CHUNK), jnp.float32),
            pltpu.SemaphoreType.DMA,
            pltpu.SemaphoreType.DMA,
        ),
    )
    def sc_kernel(idx_hbm, v_hbm, w_hbm, s_out, q_out, w_out,
                  idxf, idx_all, vrows, wvals, s_stage, q_stage, w_stage,
                  gsem, osem):
        wid = lax.axis_index("s") * NC + lax.axis_index("c")
        row0 = wid * ROWS_PER_W
        pltpu.sync_copy(
            idx_hbm.at[pl.ds(0, F), pl.ds(row0, ROWS_PER_W)], idxf)

        def conv_body(i, carry):
            for k in range(ROWS_PER_W // L):
                vf = idxf[i, pl.ds(k * L, L)]
                idx_all[i, pl.ds(k * L, L)] = vf.astype(jnp.int32)
            return carry

        lax.fori_loop(0, F, conv_body, 0)

        def issue_gathers(c):
            slot = c % 2
            descs = []
            for j in range(F):
                irow = idx_all.at[j].at[pl.ds(c * CHUNK, CHUNK)]
                dst_v = vrows.at[slot].at[pl.ds(j * CHUNK, CHUNK)]
                descs.append(pltpu.async_copy(v_hbm.at[irow], dst_v, gsem))
                dst_w = wvals.at[slot].at[pl.ds(j * CHUNK, CHUNK)]
                descs.append(pltpu.async_copy(w_hbm.at[irow], dst_w, gsem))
            return descs

        def compute(c):
            slot = c % 2
            vr = vrows.at[slot]
            wv = wvals.at[slot]

            def row_body(r, carry):
                vs = [vr[j * CHUNK + r, :] for j in range(F)]
                s_stage[slot, r, :] = _tree_add(vs)
                q_stage[slot, r, :] = _tree_add([v * v for v in vs])
                return carry

            lax.fori_loop(0, CHUNK, row_body, 0)

            def fo_body(j, accs):
                return tuple(
                    accs[k] + wv[pl.ds(j * CHUNK + k * L, L)]
                    for k in range(CHUNK // L))

            zero = jnp.zeros((L,), jnp.float32)
            accs = lax.fori_loop(0, F, fo_body, (zero,) * (CHUNK // L))
            for k in range(CHUNK // L):
                w_stage[slot, pl.ds(k * L, L)] = accs[k]

        def issue_out(c):
            slot = c % 2
            ob = wid * ROWS_PER_W + c * CHUNK
            return [
                pltpu.async_copy(s_stage.at[slot], s_out.at[pl.ds(ob, CHUNK)], osem),
                pltpu.async_copy(q_stage.at[slot], q_out.at[pl.ds(ob, CHUNK)], osem),
                pltpu.async_copy(w_stage.at[slot], w_out.at[pl.ds(ob, CHUNK)], osem),
            ]

        gd = {0: issue_gathers(0)}
        od = {}
        for c in range(NCHUNK):
            if c + 1 < NCHUNK:
                gd[c + 1] = issue_gathers(c + 1)
            for d in gd.pop(c):
                d.wait()
            if c - 2 in od:
                for d in od.pop(c - 2):
                    d.wait()
            compute(c)
            od[c] = issue_out(c)
        for descs in od.values():
            for d in descs:
                d.wait()

    return sc_kernel(inputs_t, v_tab, w_tab)


def _tc_combine(n2_ref, vne_ref, vn2e_ref, wne_ref, m_ref, b_ref,
                s_ref, q_ref, wv_ref, o_ref):
    x = n2_ref[...]
    sn = jnp.dot(x, vne_ref[...], preferred_element_type=jnp.float32)
    qn = jnp.dot(x * x, vn2e_ref[...], preferred_element_type=jnp.float32)
    st = s_ref[...] + sn
    ss = st * st - q_ref[...] - qn
    second = 0.5 * jnp.dot(ss, m_ref[...], preferred_element_type=jnp.float32)
    first = wv_ref[...] + jnp.dot(x, wne_ref[...], preferred_element_type=jnp.float32)
    o_ref[...] = first + second + b_ref[0, 0]


def kernel(inputs, w_one_hot, w_numeric, v_one_hot, v_numeric, b):
    inputs_t = inputs.T
    s, q, wsum = _sc_fm_stats(inputs_t, v_one_hot, w_one_hot.reshape(V))

    n2 = inputs[:, F:].reshape(B // 8, 8 * NUMERIC)
    eye8 = jnp.eye(8, dtype=jnp.float32)
    vne = jnp.kron(eye8, v_numeric)
    vn2e = jnp.kron(eye8, v_numeric * v_numeric)
    wne = jnp.kron(eye8, w_numeric)
    m = jnp.kron(eye8, jnp.ones((EMB, 1), jnp.float32))
    s_v = s.reshape(B * EMB // 128, 128)
    q_v = q.reshape(B * EMB // 128, 128)
    w_v = wsum.reshape(B // 8, 8)

    blk = 256
    g = (B // 8) // blk
    out8 = pl.pallas_call(
        _tc_combine,
        grid=(g,),
        in_specs=[
            pl.BlockSpec((blk, 8 * NUMERIC), lambda i: (i, 0)),
            pl.BlockSpec((8 * NUMERIC, 128), lambda i: (0, 0)),
            pl.BlockSpec((8 * NUMERIC, 128), lambda i: (0, 0)),
            pl.BlockSpec((8 * NUMERIC, 8), lambda i: (0, 0)),
            pl.BlockSpec((128, 8), lambda i: (0, 0)),
            pl.BlockSpec((1, 1), lambda i: (0, 0)),
            pl.BlockSpec((blk, 128), lambda i: (i, 0)),
            pl.BlockSpec((blk, 128), lambda i: (i, 0)),
            pl.BlockSpec((blk, 8), lambda i: (i, 0)),
        ],
        out_specs=pl.BlockSpec((blk, 8), lambda i: (i, 0)),
        out_shape=jax.ShapeDtypeStruct((B // 8, 8), jnp.float32),
    )(n2, vne, vn2e, wne, m, b.reshape(1, 1), s_v, q_v, w_v)
    return out8.reshape(B, 1)


# speedup vs baseline: 3.2083x; 1.0008x over previous
import functools

import jax
import jax.numpy as jnp
from jax import lax
from jax.experimental import pallas as pl
from jax.experimental.pallas import tpu as pltpu
from jax.experimental.pallas import tpu_sc as plsc

NC = 2
NS = 16
NW = NC * NS
L = 16

B = 16384
F = 26
EMB = 16
NUMERIC = 13
V = 1_000_000

ROWS_PER_W = B // NW
CHUNK = 64
NCHUNK = ROWS_PER_W // CHUNK
IDX_PER_CHUNK = CHUNK * F


def _tree_add(vs):
    while len(vs) > 1:
        nxt = [vs[i] + vs[i + 1] for i in range(0, len(vs) - 1, 2)]
        if len(vs) % 2:
            nxt.append(vs[-1])
        vs = nxt
    return vs[0]


def _sc_fm_stats(inputs_t, v_tab, w_tab):
    mesh = plsc.VectorSubcoreMesh(
        core_axis_name="c", subcore_axis_name="s",
        num_cores=NC, num_subcores=NS)

    @functools.partial(
        pl.kernel,
        out_type=(
            jax.ShapeDtypeStruct((B, EMB), jnp.float32),
            jax.ShapeDtypeStruct((B, EMB), jnp.float32),
            jax.ShapeDtypeStruct((B,), jnp.float32),
        ),
        mesh=mesh,
        compiler_params=pltpu.CompilerParams(use_tc_tiling_on_sc=False),
        scratch_types=(
            pltpu.VMEM((F, ROWS_PER_W), jnp.float32),
            pltpu.VMEM((F, ROWS_PER_W), jnp.int32),
            pltpu.VMEM((2, IDX_PER_CHUNK, EMB), jnp.float32),
            pltpu.VMEM((2, IDX_PER_CHUNK), jnp.float32),
            pltpu.VMEM((2, CHUNK, EMB), jnp.float32),
            pltpu.VMEM((2, CHUNK, EMB), jnp.float32),
            pltpu.VMEM((2, CHUNK), jnp.float32),
            pltpu.SemaphoreType.DMA,
            pltpu.SemaphoreType.DMA,
        ),
    )
    def sc_kernel(idx_hbm, v_hbm, w_hbm, s_out, q_out, w_out,
                  idxf, idx_all, vrows, wvals, s_stage, q_stage, w_stage,
                  gsem, osem):
        wid = lax.axis_index("s") * NC + lax.axis_index("c")
        row0 = wid * ROWS_PER_W
        pltpu.sync_copy(
            idx_hbm.at[pl.ds(0, F), pl.ds(row0, ROWS_PER_W)], idxf)

        def conv_body(i, carry):
            for k in range(ROWS_PER_W // L):
                vf = idxf[i, pl.ds(k * L, L)]
                idx_all[i, pl.ds(k * L, L)] = vf.astype(jnp.int32)
            return carry

        lax.fori_loop(0, F, conv_body, 0)

        def issue_gathers(c):
            slot = c % 2
            descs = []
            for j in range(F):
                irow = idx_all.at[j].at[pl.ds(c * CHUNK, CHUNK)]
                dst_v = vrows.at[slot].at[pl.ds(j * CHUNK, CHUNK)]
                descs.append(pltpu.async_copy(v_hbm.at[irow], dst_v, gsem))
                dst_w = wvals.at[slot].at[pl.ds(j * CHUNK, CHUNK)]
                descs.append(pltpu.async_copy(w_hbm.at[0].at[irow], dst_w, gsem))
            return descs

        def compute(c):
            slot = c % 2
            vr = vrows.at[slot]
            wv = wvals.at[slot]

            def row_body(r, carry):
                vs = [vr[j * CHUNK + r, :] for j in range(F)]
                s_stage[slot, r, :] = _tree_add(vs)
                q_stage[slot, r, :] = _tree_add([v * v for v in vs])
                return carry

            lax.fori_loop(0, CHUNK, row_body, 0)

            def fo_body(j, accs):
                return tuple(
                    accs[k] + wv[pl.ds(j * CHUNK + k * L, L)]
                    for k in range(CHUNK // L))

            zero = jnp.zeros((L,), jnp.float32)
            accs = lax.fori_loop(0, F, fo_body, (zero,) * (CHUNK // L))
            for k in range(CHUNK // L):
                w_stage[slot, pl.ds(k * L, L)] = accs[k]

        def issue_out(c):
            slot = c % 2
            ob = wid * ROWS_PER_W + c * CHUNK
            return [
                pltpu.async_copy(s_stage.at[slot], s_out.at[pl.ds(ob, CHUNK)], osem),
                pltpu.async_copy(q_stage.at[slot], q_out.at[pl.ds(ob, CHUNK)], osem),
                pltpu.async_copy(w_stage.at[slot], w_out.at[pl.ds(ob, CHUNK)], osem),
            ]

        gd = {0: issue_gathers(0)}
        od = {}
        for c in range(NCHUNK):
            if c + 1 < NCHUNK:
                gd[c + 1] = issue_gathers(c + 1)
            for d in gd.pop(c):
                d.wait()
            if c - 2 in od:
                for d in od.pop(c - 2):
                    d.wait()
            compute(c)
            od[c] = issue_out(c)
        for descs in od.values():
            for d in descs:
                d.wait()

    return sc_kernel(inputs_t, v_tab, w_tab)


def _tc_combine(n2_ref, vne_ref, vn2e_ref, wne_ref, m_ref, b_ref,
                s_ref, q_ref, wv_ref, o_ref):
    x = n2_ref[...]
    sn = jnp.dot(x, vne_ref[...], preferred_element_type=jnp.float32)
    qn = jnp.dot(x * x, vn2e_ref[...], preferred_element_type=jnp.float32)
    st = s_ref[...] + sn
    ss = st * st - q_ref[...] - qn
    second = 0.5 * jnp.dot(ss, m_ref[...], preferred_element_type=jnp.float32)
    first = wv_ref[...] + jnp.dot(x, wne_ref[...], preferred_element_type=jnp.float32)
    o_ref[...] = first + second + b_ref[0, 0]


def kernel(inputs, w_one_hot, w_numeric, v_one_hot, v_numeric, b):
    inputs_t = inputs.T
    s, q, wsum = _sc_fm_stats(inputs_t, v_one_hot, w_one_hot.T)

    n2 = inputs[:, F:].reshape(B // 8, 8 * NUMERIC)
    eye8 = jnp.eye(8, dtype=jnp.float32)
    vne = jnp.kron(eye8, v_numeric)
    vn2e = jnp.kron(eye8, v_numeric * v_numeric)
    wne = jnp.kron(eye8, w_numeric)
    m = jnp.kron(eye8, jnp.ones((EMB, 1), jnp.float32))
    s_v = s.reshape(B * EMB // 128, 128)
    q_v = q.reshape(B * EMB // 128, 128)
    w_v = wsum.reshape(B // 8, 8)

    blk = 256
    g = (B // 8) // blk
    out8 = pl.pallas_call(
        _tc_combine,
        grid=(g,),
        in_specs=[
            pl.BlockSpec((blk, 8 * NUMERIC), lambda i: (i, 0)),
            pl.BlockSpec((8 * NUMERIC, 128), lambda i: (0, 0)),
            pl.BlockSpec((8 * NUMERIC, 128), lambda i: (0, 0)),
            pl.BlockSpec((8 * NUMERIC, 8), lambda i: (0, 0)),
            pl.BlockSpec((128, 8), lambda i: (0, 0)),
            pl.BlockSpec((1, 1), lambda i: (0, 0)),
            pl.BlockSpec((blk, 128), lambda i: (i, 0)),
            pl.BlockSpec((blk, 128), lambda i: (i, 0)),
            pl.BlockSpec((blk, 8), lambda i: (i, 0)),
        ],
        out_specs=pl.BlockSpec((blk, 8), lambda i: (i, 0)),
        out_shape=jax.ShapeDtypeStruct((B // 8, 8), jnp.float32),
    )(n2, vne, vn2e, wne, m, b.reshape(1, 1), s_v, q_v, w_v)
    return out8.reshape(B, 1)
